# Initial kernel scaffold; baseline (speedup 1.0000x reference)
import jax
import jax.numpy as jnp
from jax.experimental import pallas as pl


def kernel(x, edge_index, class_labels, W1, b1, W2, b2):
    raise NotImplementedError("write your pallas kernel here")


# speedup vs baseline: 12.2108x; 12.2108x over previous
import functools

import jax
import jax.numpy as jnp
from jax import lax
from jax.experimental import pallas as pl
from jax.experimental.pallas import tpu as pltpu
from jax.experimental.pallas import tpu_sc as plsc

N_NODES = 10000
N_PAD = 10240
N_EDGES = 320000
D_IN = 128
C_CLS = 10
C_PAD = 16
NC = 2
NS = 16
NW = NC * NS
EPT = N_EDGES // NW
ECH = 128
NCH = -(-EPT // ECH)
EPT_PAD = NCH * ECH
RPS = N_PAD // NS
BN = 1024


def _mesh():
    return plsc.VectorSubcoreMesh(core_axis_name="c", subcore_axis_name="s")


def _sc_degree(cols3, zeros128):
    @functools.partial(
        pl.kernel,
        mesh=_mesh(),
        out_type=jax.ShapeDtypeStruct((NC * N_PAD, 128), jnp.float32),
        scratch_types=[
            pltpu.VMEM((NCH, ECH), jnp.int32),
            pltpu.VMEM((ECH, 128), jnp.float32),
            pltpu.VMEM_SHARED((N_PAD, 128), jnp.float32),
        ],
    )
    def k(cols_hbm, z_hbm, out_hbm, cols_v, ones_v, acc):
        c = lax.axis_index("c")
        s = lax.axis_index("s")
        wid = s * NC + c
        one_vec = jnp.full((16,), 1.0, jnp.float32)

        def fill_ones(i, carry):
            for j in range(8):
                ones_v[i, pl.ds(j * 16, 16)] = one_vec
            return carry

        lax.fori_loop(0, ECH, fill_ones, 0)
        pltpu.sync_copy(z_hbm, acc.at[pl.ds(s * RPS, RPS)])
        pltpu.sync_copy(cols_hbm.at[wid], cols_v)
        plsc.subcore_barrier()

        def body(j, carry):
            pltpu.sync_copy(ones_v, acc.at[cols_v.at[j]], add=True)
            return carry

        lax.fori_loop(0, NCH, body, 0)
        plsc.subcore_barrier()
        base = c * N_PAD + s * RPS
        pltpu.sync_copy(acc.at[pl.ds(s * RPS, RPS)], out_hbm.at[pl.ds(base, RPS)])

    return k(cols3, zeros128)


def _sc_scatter(g, rows3, cols3, zeros128):
    @functools.partial(
        pl.kernel,
        mesh=_mesh(),
        out_type=jax.ShapeDtypeStruct((NC * N_PAD, 128), jnp.float32),
        scratch_types=[
            pltpu.VMEM((NCH, ECH), jnp.int32),
            pltpu.VMEM((NCH, ECH), jnp.int32),
            pltpu.VMEM((ECH, 128), jnp.float32),
            pltpu.VMEM_SHARED((N_PAD, 128), jnp.float32),
            pltpu.SemaphoreType.DMA,
        ],
    )
    def k(g_hbm, rows_hbm, cols_hbm, z_hbm, out_hbm, rows_v, cols_v, buf, acc, sem):
        c = lax.axis_index("c")
        s = lax.axis_index("s")
        wid = s * NC + c
        pltpu.sync_copy(z_hbm, acc.at[pl.ds(s * RPS, RPS)])
        pltpu.sync_copy(rows_hbm.at[wid], rows_v)
        pltpu.sync_copy(cols_hbm.at[wid], cols_v)
        plsc.subcore_barrier()

        def body(j, carry):
            pltpu.async_copy(g_hbm.at[rows_v.at[j]], buf, sem).wait()
            pltpu.sync_copy(buf, acc.at[cols_v.at[j]], add=True)
            return carry

        lax.fori_loop(0, NCH, body, 0)
        plsc.subcore_barrier()
        base = c * N_PAD + s * RPS
        pltpu.sync_copy(acc.at[pl.ds(s * RPS, RPS)], out_hbm.at[pl.ds(base, RPS)])

    return k(g, rows3, cols3, zeros128)


def _dis(d_ref):
    deg = d_ref[0] + d_ref[1] + 1.0
    return lax.rsqrt(deg)


def _tc_layer1(x_pad, lab2, degp, W1a, W1b):
    def body(x_ref, lab_ref, d_ref, wa_ref, wb_ref, out_ref):
        dis = _dis(d_ref)
        h = jnp.dot(x_ref[...], wa_ref[...], preferred_element_type=jnp.float32)
        iota = lax.broadcasted_iota(jnp.int32, (BN, C_PAD), 1)
        oh = (lab_ref[...] == iota).astype(jnp.float32)
        h = h + jnp.dot(oh, wb_ref[...], preferred_element_type=jnp.float32)
        out_ref[...] = dis * h

    return pl.pallas_call(
        body,
        grid=(N_PAD // BN,),
        in_specs=[
            pl.BlockSpec((BN, 128), lambda i: (i, 0)),
            pl.BlockSpec((BN, 1), lambda i: (i, 0)),
            pl.BlockSpec((NC, BN, 1), lambda i: (0, i, 0)),
            pl.BlockSpec((128, 128), lambda i: (0, 0)),
            pl.BlockSpec((C_PAD, 128), lambda i: (0, 0)),
        ],
        out_specs=pl.BlockSpec((BN, 128), lambda i: (i, 0)),
        out_shape=jax.ShapeDtypeStruct((N_PAD, 128), jnp.float32),
    )(x_pad, lab2, degp, W1a, W1b)


def _tc_layer2(s1, g1, degp, W2, b1):
    def body(s_ref, g_ref, d_ref, w_ref, b_ref, out_ref):
        dis = _dis(d_ref)
        z = dis * (s_ref[0] + s_ref[1] + g_ref[...]) + b_ref[...]
        h = jnp.maximum(z, 0.0)
        out_ref[...] = dis * jnp.dot(h, w_ref[...], preferred_element_type=jnp.float32)

    return pl.pallas_call(
        body,
        grid=(N_PAD // BN,),
        in_specs=[
            pl.BlockSpec((NC, BN, 128), lambda i: (0, i, 0)),
            pl.BlockSpec((BN, 128), lambda i: (i, 0)),
            pl.BlockSpec((NC, BN, 1), lambda i: (0, i, 0)),
            pl.BlockSpec((128, 128), lambda i: (0, 0)),
            pl.BlockSpec((1, 128), lambda i: (0, 0)),
        ],
        out_specs=pl.BlockSpec((BN, 128), lambda i: (i, 0)),
        out_shape=jax.ShapeDtypeStruct((N_PAD, 128), jnp.float32),
    )(s1, g1, degp, W2, b1)


def _tc_layer3(s2, g2, degp, b2):
    def body(s_ref, g_ref, d_ref, b_ref, out_ref):
        dis = _dis(d_ref)
        out_ref[...] = dis * (s_ref[0] + s_ref[1] + g_ref[...]) + b_ref[...]

    return pl.pallas_call(
        body,
        grid=(N_PAD // BN,),
        in_specs=[
            pl.BlockSpec((NC, BN, 128), lambda i: (0, i, 0)),
            pl.BlockSpec((BN, 128), lambda i: (i, 0)),
            pl.BlockSpec((NC, BN, 1), lambda i: (0, i, 0)),
            pl.BlockSpec((1, 128), lambda i: (0, 0)),
        ],
        out_specs=pl.BlockSpec((BN, 128), lambda i: (i, 0)),
        out_shape=jax.ShapeDtypeStruct((N_PAD, 128), jnp.float32),
    )(s2, g2, degp, b2)


def kernel(x, edge_index, class_labels, W1, b1, W2, b2):
    row = edge_index[0]
    col = edge_index[1]
    pad_e = EPT_PAD - EPT
    rows3 = jnp.pad(row.reshape(NW, EPT), ((0, 0), (0, pad_e))).reshape(
        NW, NCH, ECH)
    cols3 = jnp.pad(col.reshape(NW, EPT), ((0, 0), (0, pad_e)),
                    constant_values=N_PAD - 1).reshape(NW, NCH, ECH)
    x_pad = jnp.pad(x, ((0, N_PAD - N_NODES), (0, 0)))
    lab2 = jnp.pad(class_labels, (0, N_PAD - N_NODES)).reshape(N_PAD, 1)
    W1a = W1[:D_IN]
    W1b = jnp.pad(W1[D_IN:], ((0, C_PAD - C_CLS), (0, 0)))
    zeros128 = jnp.zeros((RPS, 128), jnp.float32)

    degp = _sc_degree(cols3, zeros128).reshape(NC, N_PAD, 128)[:, :, :1]
    g1 = _tc_layer1(x_pad, lab2, degp, W1a, W1b)
    s1 = _sc_scatter(g1, rows3, cols3, zeros128).reshape(NC, N_PAD, 128)
    g2 = _tc_layer2(s1, g1, degp, W2, b1.reshape(1, 128))
    s2 = _sc_scatter(g2, rows3, cols3, zeros128).reshape(NC, N_PAD, 128)
    out = _tc_layer3(s2, g2, degp, b2.reshape(1, 128))
    return out[:N_NODES]
